# baseline (speedup 1.0000x reference)
import jax
import jax.numpy as jnp
import numpy as np
from jax.experimental import pallas as pl

_BNSCALE = float(1.0 / np.sqrt(1.0 + 1e-5))
_G = 1568


def _rearrange_imgs(a, b):
    t = jnp.stack([a, b]).reshape(2, 32, 7, 2, 2, 7, 2, 2)
    t = t.transpose(0, 4, 7, 3, 6, 2, 5, 1)
    return t.reshape(2, 4, 4 * _G)


def _kron4(W, Ci, Co):
    Z = jnp.zeros((Co, Ci), dtype=jnp.float32)
    rows = []
    for po in range(4):
        rows.append(jnp.concatenate([W if pi == po else Z for pi in range(4)],
                                    axis=1))
    return jnp.concatenate(rows, axis=0)


def _blockmat(w3, C):
    rows = []
    for oi in range(2):
        for oj in range(2):
            blocks = []
            for ii in range(2):
                for ij in range(2):
                    t = (ii - oi + 1) * 3 + (ij - oj + 1)
                    blocks.append(w3[t * C:(t + 1) * C, :])
            rows.append(jnp.concatenate(blocks, axis=1))
    return jnp.concatenate(rows, axis=0)


def _body(xmd,
          wc1, wd11, w21, wd31, wc2, wd12, w22, wd32,
          vv, fcw, fcb, out):
    voff = [0]
    for c in (16, 32, 32, 16, 8, 16, 16, 8):
        voff.append(voff[-1] + 3 * c)
    vrow = vv[...]

    def lay(K, h, li, C):
        o = voff[li]
        b = vrow[:, o:o + C]
        g = vrow[:, o + C:o + 2 * C]
        e = vrow[:, o + 2 * C:o + 3 * C]
        bt = jnp.concatenate([b] * 4, axis=1).T
        gt = jnp.concatenate([g] * 4, axis=1).T * _BNSCALE
        et = jnp.concatenate([e] * 4, axis=1).T
        z = jnp.dot(K, h, preferred_element_type=jnp.float32) + bt
        return gt * jnp.maximum(z, 0.0) + et

    xd = xmd[0]
    md = xmd[1]
    m1 = jnp.max(md, axis=0, keepdims=True)
    a1 = m1 > 0.25
    a2 = jnp.maximum(jnp.maximum(m1[:, 0:_G], m1[:, _G:2 * _G]),
                     jnp.maximum(m1[:, 2 * _G:3 * _G], m1[:, 3 * _G:4 * _G])) > 0.25

    x1 = lay(_kron4(wc1[...].T, 1, 16), xd, 0, 16)
    h = lay(_kron4(wd11[...], 16, 32), x1, 1, 32)
    h = lay(_blockmat(w21, 32), h, 2, 32)
    h = lay(_kron4(wd31[...], 32, 16), h, 3, 16)
    o = jnp.where(a1, h, x1)
    p = jnp.maximum(jnp.maximum(o[0:16], o[16:32]),
                    jnp.maximum(o[32:48], o[48:64]))
    p = jnp.concatenate([p[:, 0:_G], p[:, _G:2 * _G],
                         p[:, 2 * _G:3 * _G], p[:, 3 * _G:4 * _G]],
                        axis=0)

    x2 = lay(_kron4(wc2[...], 16, 8), p, 4, 8)
    h = lay(_kron4(wd12[...], 8, 16), x2, 5, 16)
    h = lay(_blockmat(w22, 16), h, 6, 16)
    h = lay(_kron4(wd32[...], 16, 8), h, 7, 8)
    o = jnp.where(a2, h, x2)
    q = jnp.maximum(jnp.maximum(o[0:8], o[8:16]),
                    jnp.maximum(o[16:24], o[24:32]))
    Q = jnp.concatenate([q[:, k * 32:(k + 1) * 32] for k in range(49)],
                        axis=0)

    lg = jnp.dot(fcw[...], Q, preferred_element_type=jnp.float32) + fcb[...].T
    mx = jnp.max(lg, axis=0, keepdims=True)
    e = jnp.exp(lg - mx)
    out[...] = (e / jnp.sum(e, axis=0, keepdims=True)).T


def _wmat(w):
    if w.shape[1] == 1:
        return w.reshape(1, w.shape[0])
    return w.reshape(w.shape[0], w.shape[1])


def _w3x3(w):
    return w.transpose(2, 3, 0, 1).reshape(-1, w.shape[1])


def _vec(v):
    return v.reshape(1, -1)


def kernel(x, mask1, params):
    xmd = _rearrange_imgs(x, mask1)

    p1, p2 = params['srb1'], params['srb2']
    ops = [xmd]
    vparts = []
    for p in (p1, p2):
        ops += [_wmat(p['cw']), _wmat(p['d1w']), _w3x3(p['d2w']),
                _wmat(p['d3w'])]
        for k in ('c', 'd1', 'd2', 'd3'):
            vparts += [p[k + 'b'], p[k + 'g'], p[k + 'be']]
    ops.append(jnp.concatenate(vparts).reshape(1, -1))
    ops += [params['fc_w'].reshape(10, 8, 49).transpose(0, 2, 1).reshape(10, 392),
            params['fc_b'].reshape(1, -1)]

    return pl.pallas_call(
        _body,
        out_shape=jax.ShapeDtypeStruct((32, 10), jnp.float32),
    )(*ops)


# speedup vs baseline: 1.0245x; 1.0245x over previous
import jax
import jax.numpy as jnp
import numpy as np
from jax.experimental import pallas as pl

_BNSCALE = float(1.0 / np.sqrt(1.0 + 1e-5))
_G = 1568


def _rearrange_img(img):
    t = img.reshape(32, 7, 2, 2, 7, 2, 2)
    t = t.transpose(3, 6, 2, 5, 1, 4, 0)
    return t.reshape(4, 4 * _G)


def _kron4(W, Ci, Co):
    Z = jnp.zeros((Co, Ci), dtype=jnp.float32)
    rows = []
    for po in range(4):
        rows.append(jnp.concatenate([W if pi == po else Z for pi in range(4)],
                                    axis=1))
    return jnp.concatenate(rows, axis=0)


def _blockmat(w3, C):
    rows = []
    for oi in range(2):
        for oj in range(2):
            blocks = []
            for ii in range(2):
                for ij in range(2):
                    t = (ii - oi + 1) * 3 + (ij - oj + 1)
                    blocks.append(w3[t * C:(t + 1) * C, :])
            rows.append(jnp.concatenate(blocks, axis=1))
    return jnp.concatenate(rows, axis=0)


def _body(xd, md,
          wc1, bc1, gc1, ec1, wd11, bd11, gd11, ed11,
          w21, bd21, gd21, ed21, wd31, bd31, gd31, ed31,
          wc2, bc2, gc2, ec2, wd12, bd12, gd12, ed12,
          w22, bd22, gd22, ed22, wd32, bd32, gd32, ed32,
          fcw, fcb, out):
    def lay(K, h, b, g, e):
        bt = jnp.concatenate([b[...]] * 4, axis=1).T
        gt = jnp.concatenate([g[...]] * 4, axis=1).T * _BNSCALE
        et = jnp.concatenate([e[...]] * 4, axis=1).T
        z = jnp.dot(K, h, preferred_element_type=jnp.float32) + bt
        return gt * jnp.maximum(z, 0.0) + et

    m1 = jnp.max(md[...], axis=0, keepdims=True)
    a1 = m1 > 0.25
    a2 = jnp.maximum(jnp.maximum(m1[:, 0:_G], m1[:, _G:2 * _G]),
                     jnp.maximum(m1[:, 2 * _G:3 * _G], m1[:, 3 * _G:4 * _G])) > 0.25

    x1 = lay(_kron4(wc1[...].T, 1, 16), xd[...], bc1, gc1, ec1)
    h = lay(_kron4(wd11[...], 16, 32), x1, bd11, gd11, ed11)
    h = lay(_blockmat(w21, 32), h, bd21, gd21, ed21)
    h = lay(_kron4(wd31[...], 32, 16), h, bd31, gd31, ed31)
    o = jnp.where(a1, h, x1)
    p = jnp.maximum(jnp.maximum(o[0:16], o[16:32]),
                    jnp.maximum(o[32:48], o[48:64]))
    p = jnp.concatenate([p[:, 0:_G], p[:, _G:2 * _G],
                         p[:, 2 * _G:3 * _G], p[:, 3 * _G:4 * _G]],
                        axis=0)

    x2 = lay(_kron4(wc2[...], 16, 8), p, bc2, gc2, ec2)
    h = lay(_kron4(wd12[...], 8, 16), x2, bd12, gd12, ed12)
    h = lay(_blockmat(w22, 16), h, bd22, gd22, ed22)
    h = lay(_kron4(wd32[...], 16, 8), h, bd32, gd32, ed32)
    o = jnp.where(a2, h, x2)
    q = jnp.maximum(jnp.maximum(o[0:8], o[8:16]),
                    jnp.maximum(o[16:24], o[24:32]))
    Q = jnp.concatenate([q[:, k * 32:(k + 1) * 32] for k in range(49)],
                        axis=0)

    lg = jnp.dot(fcw[...], Q, preferred_element_type=jnp.float32) + fcb[...].T
    mx = jnp.max(lg, axis=0, keepdims=True)
    e = jnp.exp(lg - mx)
    out[...] = (e / jnp.sum(e, axis=0, keepdims=True)).T


def _wmat(w):
    if w.shape[1] == 1:
        return w.reshape(1, w.shape[0])
    return w.reshape(w.shape[0], w.shape[1])


def _w3x3(w):
    return w.transpose(2, 3, 0, 1).reshape(-1, w.shape[1])


def _vec(v):
    return v.reshape(1, -1)


def kernel(x, mask1, params):
    p1, p2 = params['srb1'], params['srb2']
    ops = [_rearrange_img(x), _rearrange_img(mask1)]
    for p in (p1, p2):
        ops += [_wmat(p['cw']), _vec(p['cb']), _vec(p['cg']), _vec(p['cbe']),
                _wmat(p['d1w']), _vec(p['d1b']), _vec(p['d1g']), _vec(p['d1be']),
                _w3x3(p['d2w']), _vec(p['d2b']), _vec(p['d2g']), _vec(p['d2be']),
                _wmat(p['d3w']), _vec(p['d3b']), _vec(p['d3g']), _vec(p['d3be'])]
    ops += [params['fc_w'].reshape(10, 8, 49).transpose(0, 2, 1).reshape(10, 392),
            params['fc_b'].reshape(1, -1)]

    return pl.pallas_call(
        _body,
        out_shape=jax.ShapeDtypeStruct((32, 10), jnp.float32),
    )(*ops)


# speedup vs baseline: 1.0790x; 1.0532x over previous
import jax
import jax.numpy as jnp
import numpy as np
from jax.experimental import pallas as pl

_BNSCALE = float(1.0 / np.sqrt(1.0 + 1e-5))
_G = 1568


def _rearrange_imgs(a, b):
    t = jnp.stack([a, b]).reshape(2, 32, 7, 2, 2, 7, 2, 2)
    t = t.transpose(0, 4, 7, 3, 6, 2, 5, 1)
    return t.reshape(2, 4, 4 * _G)


def _kron4(W, Ci, Co):
    Z = jnp.zeros((Co, Ci), dtype=jnp.float32)
    rows = []
    for po in range(4):
        rows.append(jnp.concatenate([W if pi == po else Z for pi in range(4)],
                                    axis=1))
    return jnp.concatenate(rows, axis=0)


def _blockmat(w3, C):
    rows = []
    for oi in range(2):
        for oj in range(2):
            blocks = []
            for ii in range(2):
                for ij in range(2):
                    t = (ii - oi + 1) * 3 + (ij - oj + 1)
                    blocks.append(w3[t * C:(t + 1) * C, :])
            rows.append(jnp.concatenate(blocks, axis=1))
    return jnp.concatenate(rows, axis=0)


def _body(xmd,
          wc1, bc1, gc1, ec1, wd11, bd11, gd11, ed11,
          w21, bd21, gd21, ed21, wd31, bd31, gd31, ed31,
          wc2, bc2, gc2, ec2, wd12, bd12, gd12, ed12,
          w22, bd22, gd22, ed22, wd32, bd32, gd32, ed32,
          fcw, fcb, out):
    def lay(K, h, b, g, e):
        bt = jnp.concatenate([b[...]] * 4, axis=1).T
        gt = jnp.concatenate([g[...]] * 4, axis=1).T * _BNSCALE
        et = jnp.concatenate([e[...]] * 4, axis=1).T
        z = jnp.dot(K, h, preferred_element_type=jnp.float32) + bt
        return gt * jnp.maximum(z, 0.0) + et

    xd = xmd[0]
    md = xmd[1]
    m1 = jnp.max(md, axis=0, keepdims=True)
    a1 = m1 > 0.25
    a2 = jnp.maximum(jnp.maximum(m1[:, 0:_G], m1[:, _G:2 * _G]),
                     jnp.maximum(m1[:, 2 * _G:3 * _G], m1[:, 3 * _G:4 * _G])) > 0.25

    x1 = lay(_kron4(wc1[...].T, 1, 16), xd, bc1, gc1, ec1)
    h = lay(_kron4(wd11[...], 16, 32), x1, bd11, gd11, ed11)
    h = lay(_blockmat(w21, 32), h, bd21, gd21, ed21)
    h = lay(_kron4(wd31[...], 32, 16), h, bd31, gd31, ed31)
    o = jnp.where(a1, h, x1)
    p = jnp.maximum(jnp.maximum(o[0:16], o[16:32]),
                    jnp.maximum(o[32:48], o[48:64]))
    p = jnp.concatenate([p[:, 0:_G], p[:, _G:2 * _G],
                         p[:, 2 * _G:3 * _G], p[:, 3 * _G:4 * _G]],
                        axis=0)

    x2 = lay(_kron4(wc2[...], 16, 8), p, bc2, gc2, ec2)
    h = lay(_kron4(wd12[...], 8, 16), x2, bd12, gd12, ed12)
    h = lay(_blockmat(w22, 16), h, bd22, gd22, ed22)
    h = lay(_kron4(wd32[...], 16, 8), h, bd32, gd32, ed32)
    o = jnp.where(a2, h, x2)
    q = jnp.maximum(jnp.maximum(o[0:8], o[8:16]),
                    jnp.maximum(o[16:24], o[24:32]))
    Q = jnp.concatenate([q[:, k * 32:(k + 1) * 32] for k in range(49)],
                        axis=0)

    lg = jnp.dot(fcw[...], Q, preferred_element_type=jnp.float32) + fcb[...].T
    mx = jnp.max(lg, axis=0, keepdims=True)
    e = jnp.exp(lg - mx)
    out[...] = (e / jnp.sum(e, axis=0, keepdims=True)).T


def _wmat(w):
    if w.shape[1] == 1:
        return w.reshape(1, w.shape[0])
    return w.reshape(w.shape[0], w.shape[1])


def _w3x3(w):
    return w.transpose(2, 3, 0, 1).reshape(-1, w.shape[1])


def _vec(v):
    return v.reshape(1, -1)


def kernel(x, mask1, params):
    xmd = _rearrange_imgs(x, mask1)

    p1, p2 = params['srb1'], params['srb2']
    ops = [xmd]
    for p in (p1, p2):
        ops += [_wmat(p['cw']), _vec(p['cb']), _vec(p['cg']), _vec(p['cbe']),
                _wmat(p['d1w']), _vec(p['d1b']), _vec(p['d1g']), _vec(p['d1be']),
                _w3x3(p['d2w']), _vec(p['d2b']), _vec(p['d2g']), _vec(p['d2be']),
                _wmat(p['d3w']), _vec(p['d3b']), _vec(p['d3g']), _vec(p['d3be'])]
    ops += [params['fc_w'].reshape(10, 8, 49).transpose(0, 2, 1).reshape(10, 392),
            params['fc_b'].reshape(1, -1)]

    return pl.pallas_call(
        _body,
        out_shape=jax.ShapeDtypeStruct((32, 10), jnp.float32),
    )(*ops)
